# baseline (speedup 1.0000x reference)
import functools

import jax
import jax.numpy as jnp
from jax import lax
from jax.experimental import pallas as pl
from jax.experimental.pallas import tpu as pltpu
from jax.experimental.pallas import tpu_sc as plsc

_B = 16384
_DIM = 64
_GAMMA = 12.0
_NC = 2
_NS = 16
_NW = _NC * _NS
_BW = _B // _NW
_NCHUNK = 4
_CH = _BW // _NCHUNK
_HALF = 499968
_NLO = _HALF
_PASS = 128
_NPASS = _BW // _PASS
_RPB = 16
_NGP = _PASS // _RPB


def _lane_shuffle(x, idx):
    dnums = lax.GatherDimensionNumbers(
        offset_dims=(), collapsed_slice_dims=(0,), start_index_map=(0,))
    return lax.gather(x, idx[:, None], dnums, (1,),
                      mode=lax.GatherScatterMode.PROMISE_IN_BOUNDS)


def _score_body(heads_hbm, rel_hbm, tails_hbm, lo_hbm, hi_hbm, rel2_hbm,
                out_hbm, hraw, rraw, traw, hlo, hhi, tlo, thi,
                hloR, hhiR, tloR, thiR, rR, outv, sem):
    wid = lax.axis_index("s") * _NC + lax.axis_index("c")
    base = wid * _BW
    lane = lax.iota(jnp.int32, 16)

    for c in range(_NCHUNK):
        off = base + c * _CH
        pltpu.sync_copy(heads_hbm.at[pl.ds(off, _CH)], hraw.at[c])
        pltpu.sync_copy(rel_hbm.at[pl.ds(off, _CH)], rraw.at[c])
        pltpu.sync_copy(tails_hbm.at[pl.ds(off, _CH)], traw.at[c])

    for c in range(_NCHUNK):
        for s8 in range(_CH // 16):
            sl = pl.ds(s8 * 16, 16)
            hv = hraw[c, sl]
            hlo[c, sl] = jnp.minimum(hv, _NLO - 1)
            hhi[c, sl] = jnp.maximum(hv - _NLO, 0)
            tv = traw[c, sl]
            tlo[c, sl] = jnp.minimum(tv, _NLO - 1)
            thi[c, sl] = jnp.maximum(tv - _NLO, 0)

    for p in range(_NPASS):
        copies = [
            pltpu.async_copy(lo_hbm.at[hlo.at[p]], hloR, sem),
            pltpu.async_copy(hi_hbm.at[hhi.at[p]], hhiR, sem),
            pltpu.async_copy(lo_hbm.at[tlo.at[p]], tloR, sem),
            pltpu.async_copy(hi_hbm.at[thi.at[p]], thiR, sem),
            pltpu.async_copy(rel2_hbm.at[rraw.at[p]], rR, sem),
        ]
        for cp in copies:
            cp.wait()

        def grp(g, carry):
            sub = (g % 8) * 16
            wh = jnp.clip(_NLO - hraw[p, pl.ds(sub, 16)], 0, 1).astype(jnp.float32)
            wt = jnp.clip(_NLO - traw[p, pl.ds(sub, 16)], 0, 1).astype(jnp.float32)
            z = jnp.zeros((16,), jnp.float32)
            o00, o01, o10, o11 = z, z, z, z
            for ri in range(_RPB):
                row = g * _RPB + ri
                s00, s01, s10, s11 = z, z, z, z
                for q in range(_DIM // 16):
                    sl = pl.ds(q * 16, 16)
                    rr = rR[row, sl]
                    a = hloR[row, sl] + rr
                    b = hhiR[row, sl] + rr
                    tl = tloR[row, sl]
                    th = thiR[row, sl]
                    s00 = s00 + jnp.abs(a - tl)
                    s01 = s01 + jnp.abs(a - th)
                    s10 = s10 + jnp.abs(b - tl)
                    s11 = s11 + jnp.abs(b - th)
                for sh in (8, 4, 2, 1):
                    s00 = s00 + _lane_shuffle(s00, lane ^ sh)
                    s01 = s01 + _lane_shuffle(s01, lane ^ sh)
                    s10 = s10 + _lane_shuffle(s10, lane ^ sh)
                    s11 = s11 + _lane_shuffle(s11, lane ^ sh)
                m = lane == ri
                o00 = jnp.where(m, s00, o00)
                o01 = jnp.where(m, s01, o01)
                o10 = jnp.where(m, s10, o10)
                o11 = jnp.where(m, s11, o11)
            b0 = o10 + (o00 - o10) * wh
            b1 = o11 + (o01 - o11) * wh
            out16 = b1 + (b0 - b1) * wt
            outv[pl.ds(p * _PASS + g * _RPB, _RPB)] = _GAMMA - out16
            return carry

        lax.fori_loop(0, _NGP, grp, 0)

    pltpu.sync_copy(outv, out_hbm.at[pl.ds(base, _BW)])


@functools.partial(
    pl.kernel,
    out_type=jax.ShapeDtypeStruct((_B,), jnp.float32),
    mesh=plsc.VectorSubcoreMesh(core_axis_name="c", subcore_axis_name="s"),
    compiler_params=pltpu.CompilerParams(use_tc_tiling_on_sc=True),
    scratch_types=[
        pltpu.VMEM((_NCHUNK, _CH), jnp.int32),
        pltpu.VMEM((_NCHUNK, _CH), jnp.int32),
        pltpu.VMEM((_NCHUNK, _CH), jnp.int32),
        pltpu.VMEM((_NCHUNK, _CH), jnp.int32),
        pltpu.VMEM((_NCHUNK, _CH), jnp.int32),
        pltpu.VMEM((_NCHUNK, _CH), jnp.int32),
        pltpu.VMEM((_NCHUNK, _CH), jnp.int32),
        pltpu.VMEM((_PASS, 2 * _DIM), jnp.float32),
        pltpu.VMEM((_PASS, 2 * _DIM), jnp.float32),
        pltpu.VMEM((_PASS, 2 * _DIM), jnp.float32),
        pltpu.VMEM((_PASS, 2 * _DIM), jnp.float32),
        pltpu.VMEM((_PASS, 2 * _DIM), jnp.float32),
        pltpu.VMEM((_BW,), jnp.float32),
        pltpu.SemaphoreType.DMA,
    ],
)
def _score(*refs):
    _score_body(*refs)


def kernel(heads, relations, tails, entity_embedding, relation_embedding):
    lo2 = jnp.pad(entity_embedding[:_HALF], ((0, 0), (0, _DIM)))
    hi2 = jnp.pad(entity_embedding[_HALF:], ((0, 0), (0, _DIM)))
    rel2 = jnp.pad(relation_embedding, ((0, 0), (0, _DIM)))
    return _score(heads.astype(jnp.int32), relations.astype(jnp.int32),
                  tails.astype(jnp.int32), lo2, hi2, rel2)


# speedup vs baseline: 2.3637x; 2.3637x over previous
import functools

import jax
import jax.numpy as jnp
from jax import lax
from jax.experimental import pallas as pl
from jax.experimental.pallas import tpu as pltpu
from jax.experimental.pallas import tpu_sc as plsc

_B = 16384
_DIM = 64
_GAMMA = 12.0
_NC = 2
_NS = 16
_NW = _NC * _NS
_BW = _B // _NW
_NCHUNK = 4
_CH = _BW // _NCHUNK
_PASS = 256
_CPP = _PASS // _CH
_NPASS = _BW // _PASS
_RPB = 16
_NGP = _PASS // _RPB


def _lane_shuffle(x, idx):
    dnums = lax.GatherDimensionNumbers(
        offset_dims=(), collapsed_slice_dims=(0,), start_index_map=(0,))
    return lax.gather(x, idx[:, None], dnums, (1,),
                      mode=lax.GatherScatterMode.PROMISE_IN_BOUNDS)


def _score_body(heads_hbm, rel_hbm, tails_hbm, ent2_hbm, rel2_hbm, out_hbm,
                hraw, rraw, traw, h2, t2, r2, outv, sem):
    wid = lax.axis_index("s") * _NC + lax.axis_index("c")
    base = wid * _BW
    lane = lax.iota(jnp.int32, 16)

    for c in range(_NCHUNK):
        off = base + c * _CH
        pltpu.sync_copy(heads_hbm.at[pl.ds(off, _CH)], hraw.at[c])
        pltpu.sync_copy(rel_hbm.at[pl.ds(off, _CH)], rraw.at[c])
        pltpu.sync_copy(tails_hbm.at[pl.ds(off, _CH)], traw.at[c])

    for p in range(_NPASS):
        copies = []
        for c2 in range(_CPP):
            c = p * _CPP + c2
            dst = pl.ds(c2 * _CH, _CH)
            copies.append(pltpu.async_copy(ent2_hbm.at[hraw.at[c]], h2.at[dst], sem))
            copies.append(pltpu.async_copy(ent2_hbm.at[traw.at[c]], t2.at[dst], sem))
            copies.append(pltpu.async_copy(rel2_hbm.at[rraw.at[c]], r2.at[dst], sem))
        for cp in copies:
            cp.wait()

        def grp(g, carry):
            out16 = jnp.zeros((16,), jnp.float32)
            for ri in range(_RPB):
                row = g * _RPB + ri
                s = jnp.zeros((16,), jnp.float32)
                for q in range(_DIM // 16):
                    sl = pl.ds(q * 16, 16)
                    s = s + jnp.abs(h2[row, sl] + r2[row, sl] - t2[row, sl])
                for sh in (8, 4, 2, 1):
                    s = s + _lane_shuffle(s, lane ^ sh)
                out16 = jnp.where(lane == ri, s, out16)
            outv[pl.ds(p * _PASS + g * _RPB, _RPB)] = _GAMMA - out16
            return carry

        lax.fori_loop(0, _NGP, grp, 0)

    pltpu.sync_copy(outv, out_hbm.at[pl.ds(base, _BW)])


@functools.partial(
    pl.kernel,
    out_type=jax.ShapeDtypeStruct((_B,), jnp.float32),
    mesh=plsc.VectorSubcoreMesh(core_axis_name="c", subcore_axis_name="s"),
    compiler_params=pltpu.CompilerParams(use_tc_tiling_on_sc=True),
    scratch_types=[
        pltpu.VMEM((_NCHUNK, _CH), jnp.int32),
        pltpu.VMEM((_NCHUNK, _CH), jnp.int32),
        pltpu.VMEM((_NCHUNK, _CH), jnp.int32),
        pltpu.VMEM((_PASS, 2 * _DIM), jnp.float32),
        pltpu.VMEM((_PASS, 2 * _DIM), jnp.float32),
        pltpu.VMEM((_PASS, 2 * _DIM), jnp.float32),
        pltpu.VMEM((_BW,), jnp.float32),
        pltpu.SemaphoreType.DMA,
    ],
)
def _score(*refs):
    _score_body(*refs)


def kernel(heads, relations, tails, entity_embedding, relation_embedding):
    ent2 = jnp.pad(entity_embedding, ((0, 0), (0, _DIM)))
    rel2 = jnp.pad(relation_embedding, ((0, 0), (0, _DIM)))
    return _score(heads.astype(jnp.int32), relations.astype(jnp.int32),
                  tails.astype(jnp.int32), ent2, rel2)
